# initial kernel scaffold (unmeasured)
import jax
import jax.numpy as jnp
from jax import lax
from jax.experimental import pallas as pl
from jax.experimental.pallas import tpu as pltpu

N_DEV = 16


def kernel(x, w_mat):
    m_per, k = x.shape
    n = w_mat.shape[1]
    n_per = n // N_DEV
    m = m_per * N_DEV

    def body(x_ref, w_ref, out_ref, y_ref, gather_ref, send_sems, recv_sems):
        p = lax.axis_index("i")

        y = jnp.dot(x_ref[...], w_ref[...], preferred_element_type=jnp.float32)
        y_ref[...] = jnp.maximum(y, 0.0).astype(jnp.bfloat16)

        gather_ref[pl.ds(p * m_per, m_per), :] = y_ref[:, pl.ds(p * n_per, n_per)]

        for d in range(1, N_DEV):
            dst = lax.rem(p + d, N_DEV)
            rdma = pltpu.make_async_remote_copy(
                src_ref=y_ref.at[:, pl.ds(dst * n_per, n_per)],
                dst_ref=gather_ref.at[pl.ds(p * m_per, m_per), :],
                send_sem=send_sems.at[d - 1],
                recv_sem=recv_sems.at[d - 1],
                device_id=(dst,),
                device_id_type=pl.DeviceIdType.MESH,
            )
            rdma.start()

        for d in range(1, N_DEV):
            src = lax.rem(p - d + N_DEV, N_DEV)
            recv = pltpu.make_async_remote_copy(
                src_ref=y_ref.at[:, pl.ds(src * n_per, n_per)],
                dst_ref=gather_ref.at[pl.ds(src * m_per, m_per), :],
                send_sem=send_sems.at[d - 1],
                recv_sem=recv_sems.at[d - 1],
                device_id=(src,),
                device_id_type=pl.DeviceIdType.MESH,
            )
            recv.wait_recv()
        for d in range(1, N_DEV):
            dst = lax.rem(p + d, N_DEV)
            send = pltpu.make_async_remote_copy(
                src_ref=y_ref.at[:, pl.ds(dst * n_per, n_per)],
                dst_ref=gather_ref.at[pl.ds(p * m_per, m_per), :],
                send_sem=send_sems.at[d - 1],
                recv_sem=recv_sems.at[d - 1],
                device_id=(dst,),
                device_id_type=pl.DeviceIdType.MESH,
            )
            send.wait_send()

        out_ref[...] = gather_ref[...].astype(jnp.float32)

    return pl.pallas_call(
        body,
        out_shape=jax.ShapeDtypeStruct((m, n_per), jnp.float32),
        in_specs=[
            pl.BlockSpec(memory_space=pltpu.VMEM),
            pl.BlockSpec(memory_space=pltpu.VMEM),
        ],
        out_specs=pl.BlockSpec(memory_space=pltpu.VMEM),
        scratch_shapes=[
            pltpu.VMEM((m_per, n), jnp.bfloat16),
            pltpu.VMEM((m, n_per), jnp.bfloat16),
            pltpu.SemaphoreType.DMA((N_DEV - 1,)),
            pltpu.SemaphoreType.DMA((N_DEV - 1,)),
        ],
        compiler_params=pltpu.CompilerParams(collective_id=0),
    )(x, w_mat)


# baseline (device time: 25058 ns/iter reference)
import jax
import jax.numpy as jnp
from jax import lax
from jax.experimental import pallas as pl
from jax.experimental.pallas import tpu as pltpu

N_DEV = 16


def kernel(x, w_mat):
    m_per, k = x.shape
    n = w_mat.shape[1]
    n_per = n // N_DEV
    m = m_per * N_DEV

    def body(x_ref, w_ref, out_ref, y_ref, gather_ref, send_sems, recv_sems):
        p = lax.axis_index("i")

        y = jnp.dot(x_ref[...], w_ref[...], preferred_element_type=jnp.float32)
        y_ref[...] = jnp.maximum(y, 0.0).astype(jnp.bfloat16)

        gather_ref[pl.ds(p * m_per, m_per), :] = y_ref[:, pl.ds(p * n_per, n_per)]

        for d in range(1, N_DEV):
            dst = lax.rem(p + d, N_DEV)
            rdma = pltpu.make_async_remote_copy(
                src_ref=y_ref.at[:, pl.ds(dst * n_per, n_per)],
                dst_ref=gather_ref.at[pl.ds(p * m_per, m_per), :],
                send_sem=send_sems.at[d - 1],
                recv_sem=recv_sems.at[d - 1],
                device_id=(dst,),
                device_id_type=pl.DeviceIdType.MESH,
            )
            rdma.start()

        for d in range(1, N_DEV):
            src = lax.rem(p - d + N_DEV, N_DEV)
            recv = pltpu.make_async_remote_copy(
                src_ref=y_ref.at[:, pl.ds(src * n_per, n_per)],
                dst_ref=gather_ref.at[pl.ds(src * m_per, m_per), :],
                send_sem=send_sems.at[d - 1],
                recv_sem=recv_sems.at[d - 1],
                device_id=(src,),
                device_id_type=pl.DeviceIdType.MESH,
            )
            recv.wait_recv()
        for d in range(1, N_DEV):
            dst = lax.rem(p + d, N_DEV)
            send = pltpu.make_async_remote_copy(
                src_ref=y_ref.at[:, pl.ds(dst * n_per, n_per)],
                dst_ref=gather_ref.at[pl.ds(p * m_per, m_per), :],
                send_sem=send_sems.at[d - 1],
                recv_sem=recv_sems.at[d - 1],
                device_id=(dst,),
                device_id_type=pl.DeviceIdType.MESH,
            )
            send.wait_send()

        out_ref[...] = gather_ref[...].astype(jnp.float32)

    return pl.pallas_call(
        body,
        out_shape=jax.ShapeDtypeStruct((m, n_per), jnp.float32),
        in_specs=[
            pl.BlockSpec(memory_space=pltpu.VMEM),
            pl.BlockSpec(memory_space=pltpu.VMEM),
        ],
        out_specs=pl.BlockSpec(memory_space=pltpu.VMEM),
        scratch_shapes=[
            pltpu.VMEM((m_per, n), jnp.bfloat16),
            pltpu.VMEM((m, n_per), jnp.bfloat16),
            pltpu.SemaphoreType.DMA((N_DEV - 1,)),
            pltpu.SemaphoreType.DMA((N_DEV - 1,)),
        ],
    )(x, w_mat)


# device time: 23997 ns/iter; 1.0442x vs baseline; 1.0442x over previous
import jax
import jax.numpy as jnp
from jax import lax
from jax.experimental import pallas as pl
from jax.experimental.pallas import tpu as pltpu

N_DEV = 16


def kernel(x, w_mat):
    m_per, k = x.shape
    n = w_mat.shape[1]
    n_per = n // N_DEV
    m = m_per * N_DEV

    def body(x_ref, w_hbm, out_ref, xb_ref, wbuf, send_buf, gather_ref,
             send_sems, recv_sems, wsems):
        p = lax.axis_index("i")

        barrier_sem = pltpu.get_barrier_semaphore()
        for d in range(1, N_DEV):
            peer = lax.rem(p + d, N_DEV)
            pl.semaphore_signal(barrier_sem, inc=1, device_id=(peer,),
                                device_id_type=pl.DeviceIdType.MESH)

        def w_dma(c, slot):
            dst = lax.rem(p + 1 + c, N_DEV)
            return pltpu.make_async_copy(
                w_hbm.at[:, pl.ds(dst * n_per, n_per)],
                wbuf.at[slot],
                wsems.at[slot],
            )

        w_dma(0, 0).start()
        xb_ref[...] = x_ref[...].astype(jnp.bfloat16)
        for c in range(N_DEV):
            slot = c % 2
            if c + 1 < N_DEV:
                w_dma(c + 1, 1 - slot).start()
            w_dma(c, slot).wait()
            z = jnp.dot(xb_ref[...], wbuf[slot].astype(jnp.bfloat16),
                        preferred_element_type=jnp.float32)
            z = jnp.maximum(z, 0.0).astype(jnp.bfloat16)
            if c < N_DEV - 1:
                dst = lax.rem(p + 1 + c, N_DEV)
                send_buf[c] = z
                if c == 0:
                    pl.semaphore_wait(barrier_sem, N_DEV - 1)
                rdma = pltpu.make_async_remote_copy(
                    src_ref=send_buf.at[c],
                    dst_ref=gather_ref.at[pl.ds(p * m_per, m_per), :],
                    send_sem=send_sems.at[c],
                    recv_sem=recv_sems.at[c],
                    device_id=(dst,),
                    device_id_type=pl.DeviceIdType.MESH,
                )
                rdma.start()
            else:
                gather_ref[pl.ds(p * m_per, m_per), :] = z

        for d in range(1, N_DEV):
            src = lax.rem(p - d + N_DEV, N_DEV)
            recv = pltpu.make_async_remote_copy(
                src_ref=send_buf.at[d - 1],
                dst_ref=gather_ref.at[pl.ds(src * m_per, m_per), :],
                send_sem=send_sems.at[d - 1],
                recv_sem=recv_sems.at[d - 1],
                device_id=(src,),
                device_id_type=pl.DeviceIdType.MESH,
            )
            recv.wait_recv()
        for d in range(1, N_DEV):
            dst = lax.rem(p + d, N_DEV)
            send = pltpu.make_async_remote_copy(
                src_ref=send_buf.at[d - 1],
                dst_ref=gather_ref.at[pl.ds(p * m_per, m_per), :],
                send_sem=send_sems.at[d - 1],
                recv_sem=recv_sems.at[d - 1],
                device_id=(dst,),
                device_id_type=pl.DeviceIdType.MESH,
            )
            send.wait_send()

        out_ref[...] = gather_ref[...].astype(jnp.float32)

    return pl.pallas_call(
        body,
        out_shape=jax.ShapeDtypeStruct((m, n_per), jnp.float32),
        in_specs=[
            pl.BlockSpec(memory_space=pltpu.VMEM),
            pl.BlockSpec(memory_space=pltpu.MemorySpace.HBM),
        ],
        out_specs=pl.BlockSpec(memory_space=pltpu.VMEM),
        scratch_shapes=[
            pltpu.VMEM((m_per, k), jnp.bfloat16),
            pltpu.VMEM((2, k, n_per), jnp.float32),
            pltpu.VMEM((N_DEV - 1, m_per, n_per), jnp.bfloat16),
            pltpu.VMEM((m, n_per), jnp.bfloat16),
            pltpu.SemaphoreType.DMA((N_DEV - 1,)),
            pltpu.SemaphoreType.DMA((N_DEV - 1,)),
            pltpu.SemaphoreType.DMA((2,)),
        ],
        compiler_params=pltpu.CompilerParams(collective_id=0),
    )(x, w_mat)


# device time: 21982 ns/iter; 1.1399x vs baseline; 1.0917x over previous
import os

import jax
import jax.numpy as jnp
from jax import lax
from jax.experimental import pallas as pl
from jax.experimental.pallas import tpu as pltpu

N_DEV = 16
_VARIANT = os.environ.get("KVARIANT", "full")


def kernel(x, w_mat):
    m_per, k = x.shape
    n = w_mat.shape[1]
    n_per = n // N_DEV
    m = m_per * N_DEV

    def body(x_ref, w_ref, out_ref, xb_ref, send_buf, gather_ref,
             send_sems, recv_sems):
        p = lax.axis_index("i")

        if _VARIANT != "nocomm":
            barrier_sem = pltpu.get_barrier_semaphore()
            for d in range(1, N_DEV):
                peer = lax.rem(p + d, N_DEV)
                pl.semaphore_signal(barrier_sem, inc=1, device_id=(peer,),
                                    device_id_type=pl.DeviceIdType.MESH)

        if _VARIANT != "f32dot":
            xb_ref[...] = x_ref[...].astype(jnp.bfloat16)

        for c in range(N_DEV):
            dst = lax.rem(p + 1 + c, N_DEV)
            wcol = w_ref[:, pl.ds(dst * n_per, n_per)]
            if _VARIANT == "onlycomm":
                z = xb_ref[:, :n_per]
            elif _VARIANT == "f32dot":
                z = jnp.dot(x_ref[...], wcol,
                            preferred_element_type=jnp.float32)
            else:
                z = jnp.dot(xb_ref[...], wcol.astype(jnp.bfloat16),
                            preferred_element_type=jnp.float32)
            z = jnp.maximum(z, 0.0).astype(jnp.bfloat16)
            if c < N_DEV - 1:
                send_buf[c] = z
                if _VARIANT == "nocomm":
                    continue
                if c == 0:
                    pl.semaphore_wait(barrier_sem, N_DEV - 1)
                rdma = pltpu.make_async_remote_copy(
                    src_ref=send_buf.at[c],
                    dst_ref=gather_ref.at[pl.ds(p * m_per, m_per), :],
                    send_sem=send_sems.at[c],
                    recv_sem=recv_sems.at[c],
                    device_id=(dst,),
                    device_id_type=pl.DeviceIdType.MESH,
                )
                rdma.start()
            else:
                gather_ref[pl.ds(p * m_per, m_per), :] = z

        for d in range(1, N_DEV) if _VARIANT != "nocomm" else []:
            src = lax.rem(p - d + N_DEV, N_DEV)
            recv = pltpu.make_async_remote_copy(
                src_ref=send_buf.at[d - 1],
                dst_ref=gather_ref.at[pl.ds(src * m_per, m_per), :],
                send_sem=send_sems.at[d - 1],
                recv_sem=recv_sems.at[d - 1],
                device_id=(src,),
                device_id_type=pl.DeviceIdType.MESH,
            )
            recv.wait_recv()
        for d in range(1, N_DEV) if _VARIANT != "nocomm" else []:
            dst = lax.rem(p + d, N_DEV)
            send = pltpu.make_async_remote_copy(
                src_ref=send_buf.at[d - 1],
                dst_ref=gather_ref.at[pl.ds(p * m_per, m_per), :],
                send_sem=send_sems.at[d - 1],
                recv_sem=recv_sems.at[d - 1],
                device_id=(dst,),
                device_id_type=pl.DeviceIdType.MESH,
            )
            send.wait_send()

        out_ref[...] = gather_ref[...].astype(jnp.float32)

    return pl.pallas_call(
        body,
        out_shape=jax.ShapeDtypeStruct((m, n_per), jnp.float32),
        in_specs=[
            pl.BlockSpec(memory_space=pltpu.VMEM),
            pl.BlockSpec(memory_space=pltpu.VMEM),
        ],
        out_specs=pl.BlockSpec(memory_space=pltpu.VMEM),
        scratch_shapes=[
            pltpu.VMEM((m_per, k), jnp.bfloat16),
            pltpu.VMEM((N_DEV - 1, m_per, n_per), jnp.bfloat16),
            pltpu.VMEM((m, n_per), jnp.bfloat16),
            pltpu.SemaphoreType.DMA((N_DEV - 1,)),
            pltpu.SemaphoreType.DMA((N_DEV - 1,)),
        ],
        compiler_params=(
            pltpu.CompilerParams(collective_id=0)
            if _VARIANT != "nocomm"
            else pltpu.CompilerParams()
        ),
    )(x, w_mat)


# device time: 21891 ns/iter; 1.1447x vs baseline; 1.0042x over previous
import jax
import jax.numpy as jnp
from jax import lax
from jax.experimental import pallas as pl
from jax.experimental.pallas import tpu as pltpu

N_DEV = 16


def kernel(x, w_mat):
    m_per, k = x.shape
    n = w_mat.shape[1]
    n_per = n // N_DEV
    m = m_per * N_DEV

    def body(x_ref, w_ref, out_ref, xb_ref, send_buf, gather_ref,
             send_sems, recv_sems):
        p = lax.axis_index("i")

        barrier_sem = pltpu.get_barrier_semaphore()
        for d in range(1, N_DEV):
            peer = lax.rem(p + d, N_DEV)
            pl.semaphore_signal(barrier_sem, inc=1, device_id=(peer,),
                                device_id_type=pl.DeviceIdType.MESH)

        xb_ref[...] = x_ref[...].astype(jnp.bfloat16)

        for c in range(N_DEV):
            dst = lax.rem(p + 1 + c, N_DEV)
            wcol = w_ref[:, pl.ds(dst * n_per, n_per)]
            z = jnp.dot(xb_ref[...], wcol.astype(jnp.bfloat16),
                        preferred_element_type=jnp.float32)
            z = jnp.maximum(z, 0.0).astype(jnp.bfloat16)
            if c < N_DEV - 1:
                send_buf[c] = z
                if c == 0:
                    pl.semaphore_wait(barrier_sem, N_DEV - 1)
                rdma = pltpu.make_async_remote_copy(
                    src_ref=send_buf.at[c],
                    dst_ref=gather_ref.at[pl.ds(p * m_per, m_per), :],
                    send_sem=send_sems.at[c],
                    recv_sem=recv_sems.at[c],
                    device_id=(dst,),
                    device_id_type=pl.DeviceIdType.MESH,
                )
                rdma.start()
            else:
                gather_ref[pl.ds(p * m_per, m_per), :] = z

        for d in range(1, N_DEV):
            src = lax.rem(p - d + N_DEV, N_DEV)
            recv = pltpu.make_async_remote_copy(
                src_ref=send_buf.at[d - 1],
                dst_ref=gather_ref.at[pl.ds(src * m_per, m_per), :],
                send_sem=send_sems.at[d - 1],
                recv_sem=recv_sems.at[d - 1],
                device_id=(src,),
                device_id_type=pl.DeviceIdType.MESH,
            )
            recv.wait_recv()
        for d in range(1, N_DEV):
            dst = lax.rem(p + d, N_DEV)
            send = pltpu.make_async_remote_copy(
                src_ref=send_buf.at[d - 1],
                dst_ref=gather_ref.at[pl.ds(p * m_per, m_per), :],
                send_sem=send_sems.at[d - 1],
                recv_sem=recv_sems.at[d - 1],
                device_id=(dst,),
                device_id_type=pl.DeviceIdType.MESH,
            )
            send.wait_send()

        out_ref[...] = gather_ref[...].astype(jnp.float32)

    return pl.pallas_call(
        body,
        out_shape=jax.ShapeDtypeStruct((m, n_per), jnp.float32),
        in_specs=[
            pl.BlockSpec(memory_space=pltpu.VMEM),
            pl.BlockSpec(memory_space=pltpu.VMEM),
        ],
        out_specs=pl.BlockSpec(memory_space=pltpu.VMEM),
        scratch_shapes=[
            pltpu.VMEM((m_per, k), jnp.bfloat16),
            pltpu.VMEM((N_DEV - 1, m_per, n_per), jnp.bfloat16),
            pltpu.VMEM((m, n_per), jnp.bfloat16),
            pltpu.SemaphoreType.DMA((N_DEV - 1,)),
            pltpu.SemaphoreType.DMA((N_DEV - 1,)),
        ],
        compiler_params=pltpu.CompilerParams(collective_id=0),
    )(x, w_mat)
